# baseline (speedup 1.0000x reference)
import functools

import jax
import jax.numpy as jnp
from jax import lax
from jax.experimental import pallas as pl
from jax.experimental.pallas import tpu as pltpu
from jax.experimental.pallas import tpu_sc as plsc

F = 26
VP1 = 100001
D = 32
B = 16384

_info = plsc.get_sparse_core_info()
NC = _info.num_cores
NS = _info.num_subcores
NW = NC * NS
BPW = B // NW
KB = 4
NG = BPW // KB
ROWS = KB * F


def _body(table_hbm, idx_hbm, out_hbm, idx_v, rows_v, out_v, sem):
    wid = lax.axis_index("s") * NC + lax.axis_index("c")
    pltpu.sync_copy(idx_hbm.at[wid], idx_v)

    def step(g, carry):
        pltpu.async_copy(table_hbm.at[idx_v.at[g]], rows_v, sem).wait()
        for b in range(KB):
            for h in range(2):
                acc = rows_v[b * F, pl.ds(h * 16, 16)]
                for f in range(1, F):
                    acc = acc + rows_v[b * F + f, pl.ds(h * 16, 16)]
                out_v[g * KB + b, pl.ds(h * 16, 16)] = acc
        return carry

    lax.fori_loop(0, NG, step, 0)
    pltpu.sync_copy(out_v, out_hbm.at[pl.ds(wid * BPW, BPW)])


@jax.jit
def _run(table_flat, idx):
    mesh = plsc.VectorSubcoreMesh(core_axis_name="c", subcore_axis_name="s")
    kfn = pl.kernel(
        _body,
        mesh=mesh,
        out_type=jax.ShapeDtypeStruct((B, D), jnp.float32),
        scratch_types=[
            pltpu.VMEM((NG, ROWS), jnp.int32),
            pltpu.VMEM((ROWS, D), jnp.float32),
            pltpu.VMEM((BPW, D), jnp.float32),
            pltpu.SemaphoreType.DMA,
        ],
        compiler_params=pltpu.CompilerParams(use_tc_tiling_on_sc=False),
    )
    return kfn(table_flat, idx)


def kernel(X, tables):
    idx = X.astype(jnp.int32) + (jnp.arange(F, dtype=jnp.int32) * VP1)[None, :]
    idx = idx.reshape(NW, NG, ROWS)
    table_flat = tables.reshape(F * VP1, D)
    return _run(table_flat, idx)


# speedup vs baseline: 32.0951x; 32.0951x over previous
import jax
import jax.numpy as jnp
from jax import lax
from jax.experimental import pallas as pl
from jax.experimental.pallas import tpu as pltpu
from jax.experimental.pallas import tpu_sc as plsc

F = 26
VOCAB = 100001
D = 32
B = 16384

_info = plsc.get_sparse_core_info()
NC = _info.num_cores
NS = _info.num_subcores
NW = NC * NS
IDXC = 8192


def _body(tab, xt, out, plane_v, idx_v, acc_v):
    d = lax.axis_index("s") * NC + lax.axis_index("c")
    for f in range(F):
        pltpu.sync_copy(tab.at[f, d], plane_v)
        for c in range(B // IDXC):
            pltpu.sync_copy(xt.at[f, pl.ds(c * IDXC, IDXC)], idx_v)

            def step(j, carry, _f=f, _c=c):
                lanes = idx_v[pl.ds(j * 16, 16)]
                vals = plsc.load_gather(plane_v, [lanes])
                sl = pl.ds(_c * IDXC + j * 16, 16)
                if _f == 0:
                    acc_v[sl] = vals
                else:
                    plsc.addupdate(acc_v.at[sl], vals)
                return carry

            lax.fori_loop(0, IDXC // 16, step, 0)
    pltpu.sync_copy(acc_v, out.at[d])


@jax.jit
def _run(tab_t, x_t):
    mesh = plsc.VectorSubcoreMesh(core_axis_name="c", subcore_axis_name="s")
    kfn = pl.kernel(
        _body,
        mesh=mesh,
        out_type=jax.ShapeDtypeStruct((D, B), jnp.float32),
        scratch_types=[
            pltpu.VMEM((VOCAB,), jnp.float32),
            pltpu.VMEM((IDXC,), jnp.int32),
            pltpu.VMEM((B,), jnp.float32),
        ],
        compiler_params=pltpu.CompilerParams(needs_layout_passes=False),
    )
    return kfn(tab_t, x_t)


def kernel(X, tables):
    x_t = X.astype(jnp.int32).T
    tab_t = jnp.transpose(tables, (0, 2, 1))
    return _run(tab_t, x_t).T


# speedup vs baseline: 39.8603x; 1.2419x over previous
import jax
import jax.numpy as jnp
from jax import lax
from jax.experimental import pallas as pl
from jax.experimental.pallas import tpu as pltpu
from jax.experimental.pallas import tpu_sc as plsc

F = 26
VOCAB = 100001
D = 32
B = 16384

_info = plsc.get_sparse_core_info()
NC = _info.num_cores
NS = _info.num_subcores
NW = NC * NS
IDXC = 8192
UNROLL = 8


def _body(tab, xt, out, plane_v, idx_v, acc_v):
    d = lax.axis_index("s") * NC + lax.axis_index("c")
    for f in range(F):
        pltpu.sync_copy(tab.at[f, d], plane_v)
        for c in range(B // IDXC):
            pltpu.sync_copy(xt.at[f, pl.ds(c * IDXC, IDXC)], idx_v)

            def step(j, carry, _f=f, _c=c):
                jb = j * (16 * UNROLL)
                for u in range(UNROLL):
                    lanes = idx_v[pl.ds(jb + u * 16, 16)]
                    vals = plsc.load_gather(plane_v, [lanes])
                    sl = pl.ds(_c * IDXC + jb + u * 16, 16)
                    if _f == 0:
                        acc_v[sl] = vals
                    else:
                        plsc.addupdate(acc_v.at[sl], vals)
                return carry

            lax.fori_loop(0, IDXC // (16 * UNROLL), step, 0)
    pltpu.sync_copy(acc_v, out.at[d])


@jax.jit
def _run(tab_t, x_t):
    mesh = plsc.VectorSubcoreMesh(core_axis_name="c", subcore_axis_name="s")
    kfn = pl.kernel(
        _body,
        mesh=mesh,
        out_type=jax.ShapeDtypeStruct((D, B), jnp.float32),
        scratch_types=[
            pltpu.VMEM((VOCAB,), jnp.float32),
            pltpu.VMEM((IDXC,), jnp.int32),
            pltpu.VMEM((B,), jnp.float32),
        ],
        compiler_params=pltpu.CompilerParams(needs_layout_passes=False),
    )
    return kfn(tab_t, x_t)


def kernel(X, tables):
    x_t = X.astype(jnp.int32).T
    tab_t = jnp.transpose(tables, (0, 2, 1))
    return _run(tab_t, x_t).T
